# baseline (speedup 1.0000x reference)
import jax
import jax.numpy as jnp
from jax import lax
from jax.experimental import pallas as pl
from jax.experimental.pallas import tpu as pltpu

_RADIUS = 2
_WIN = 2 * _RADIUS + 1
_TILE_H = 16


def _make_body(B, H, W, C):
    Cout = C - 2
    nchunk = H // _TILE_H

    def _body(idx_ref, fields_ref, out_ref, win_ref, tmpl_ref, wsem, osem):
        b = pl.program_id(0)
        nb = pl.num_programs(0)
        par = lax.rem(b, 2)

        def band_copy(bb, pp):
            py = idx_ref[bb, 0]
            return pltpu.make_async_copy(
                fields_ref.at[bb, pl.ds(py - _RADIUS, _WIN), :, :],
                win_ref.at[pp],
                wsem,
            )

        def chunk_copy(bb, pp, c):
            return pltpu.make_async_copy(
                tmpl_ref.at[pp],
                out_ref.at[bb, :, pl.ds(c * _TILE_H, _TILE_H), :],
                osem.at[pp],
            )

        @pl.when(b == 0)
        def _():
            band_copy(0, 0).start()

        @pl.when(b + 1 < nb)
        def _():
            band_copy(b + 1, 1 - par).start()

        @pl.when(b >= 2)
        def _():
            for c in range(nchunk):
                chunk_copy(b - 2, par, c).wait()

        band_copy(b, par).wait()

        px = idx_ref[b, 1]
        cx = px - _RADIUS
        ci = lax.iota(jnp.int32, W)
        cmask = ((ci >= cx) & (ci < cx + _WIN)).astype(jnp.float32)
        s1 = jnp.sum(win_ref[par], axis=0)
        s2 = jnp.sum(s1 * cmask[None, :], axis=1, keepdims=True)
        m = s2[2:, :] * (1.0 / (_WIN * _WIN))

        tmpl_ref[par] = jnp.broadcast_to(m[:, :, None], (Cout, _TILE_H, W))

        for c in range(nchunk):
            chunk_copy(b, par, c).start()

        @pl.when(b == nb - 1)
        def _():
            for c in range(nchunk):
                chunk_copy(b - 1, 1 - par, c).wait()
            for c in range(nchunk):
                chunk_copy(b, par, c).wait()

    return _body


def kernel(fields, pump_indices):
    B, H, W, C = fields.shape
    Cout = C - 2
    idx = pump_indices.astype(jnp.int32)
    fields_v = jnp.transpose(fields, (0, 1, 3, 2))

    grid_spec = pltpu.PrefetchScalarGridSpec(
        num_scalar_prefetch=1,
        grid=(B,),
        in_specs=[pl.BlockSpec(memory_space=pl.ANY)],
        out_specs=pl.BlockSpec(memory_space=pl.ANY),
        scratch_shapes=[
            pltpu.VMEM((2, _WIN, C, W), jnp.float32),
            pltpu.VMEM((2, Cout, _TILE_H, W), jnp.float32),
            pltpu.SemaphoreType.DMA,
            pltpu.SemaphoreType.DMA((2,)),
        ],
    )
    out_t = pl.pallas_call(
        _make_body(B, H, W, C),
        grid_spec=grid_spec,
        out_shape=jax.ShapeDtypeStruct((B, Cout, H, W), jnp.float32),
    )(idx, fields_v)
    return jnp.transpose(out_t, (0, 2, 3, 1))


# speedup vs baseline: 1.0006x; 1.0006x over previous
import jax
import jax.numpy as jnp
from jax import lax
from jax.experimental import pallas as pl
from jax.experimental.pallas import tpu as pltpu

_RADIUS = 2
_WIN = 2 * _RADIUS + 1
_TILE_H = 32


def _make_body(B, H, W, C):
    Cout = C - 2
    nchunk = H // _TILE_H

    def _body(idx_ref, fields_ref, out_ref, win_ref, tmpl_ref, wsem, osem):
        b = pl.program_id(0)
        nb = pl.num_programs(0)
        par = lax.rem(b, 2)

        def band_copy(bb, pp):
            py = idx_ref[bb, 0]
            return pltpu.make_async_copy(
                fields_ref.at[bb, pl.ds(py - _RADIUS, _WIN), :, :],
                win_ref.at[pp],
                wsem,
            )

        def chunk_copy(bb, pp, c):
            return pltpu.make_async_copy(
                tmpl_ref.at[pp],
                out_ref.at[bb, :, pl.ds(c * _TILE_H, _TILE_H), :],
                osem.at[pp],
            )

        @pl.when(b == 0)
        def _():
            band_copy(0, 0).start()

        @pl.when(b + 1 < nb)
        def _():
            band_copy(b + 1, 1 - par).start()

        @pl.when(b >= 2)
        def _():
            for c in range(nchunk):
                chunk_copy(b - 2, par, c).wait()

        band_copy(b, par).wait()

        px = idx_ref[b, 1]
        cx = px - _RADIUS
        ci = lax.iota(jnp.int32, W)
        cmask = ((ci >= cx) & (ci < cx + _WIN)).astype(jnp.float32)
        s1 = jnp.sum(win_ref[par], axis=0)
        s2 = jnp.sum(s1 * cmask[None, :], axis=1, keepdims=True)
        m = s2[2:, :] * (1.0 / (_WIN * _WIN))

        tmpl_ref[par] = jnp.broadcast_to(m[:, :, None], (Cout, _TILE_H, W))

        for c in range(nchunk):
            chunk_copy(b, par, c).start()

        @pl.when(b == nb - 1)
        def _():
            for c in range(nchunk):
                chunk_copy(b - 1, 1 - par, c).wait()
            for c in range(nchunk):
                chunk_copy(b, par, c).wait()

    return _body


def kernel(fields, pump_indices):
    B, H, W, C = fields.shape
    Cout = C - 2
    idx = pump_indices.astype(jnp.int32)
    fields_v = jnp.transpose(fields, (0, 1, 3, 2))

    grid_spec = pltpu.PrefetchScalarGridSpec(
        num_scalar_prefetch=1,
        grid=(B,),
        in_specs=[pl.BlockSpec(memory_space=pl.ANY)],
        out_specs=pl.BlockSpec(memory_space=pl.ANY),
        scratch_shapes=[
            pltpu.VMEM((2, _WIN, C, W), jnp.float32),
            pltpu.VMEM((2, Cout, _TILE_H, W), jnp.float32),
            pltpu.SemaphoreType.DMA,
            pltpu.SemaphoreType.DMA((2,)),
        ],
    )
    out_t = pl.pallas_call(
        _make_body(B, H, W, C),
        grid_spec=grid_spec,
        out_shape=jax.ShapeDtypeStruct((B, Cout, H, W), jnp.float32),
    )(idx, fields_v)
    return jnp.transpose(out_t, (0, 2, 3, 1))
